# baseline (speedup 1.0000x reference)
import jax
import jax.numpy as jnp
from jax.experimental import pallas as pl


def _vq_body(z_ref, cb_ref, out_ref):
    zb = z_ref[0]
    cb = cb_ref[...]
    mm = jax.lax.dot_general(
        cb, zb, (((1,), (0,)), ((), ())),
        preferred_element_type=jnp.float32)
    x2 = jnp.sum(zb * zb, axis=0, keepdims=True)
    c2 = jnp.sum(cb * cb, axis=1, keepdims=True)
    dist = (x2 - 2.0 * mm) + c2
    m = jnp.min(dist, axis=0, keepdims=True)
    ks = jax.lax.broadcasted_iota(jnp.int32, dist.shape, 0)
    big = jnp.int32(dist.shape[0])
    idx = jnp.min(jnp.where(dist == m, ks, big), axis=0)
    out_ref[0, 0, :] = idx


def kernel(z_e_x, codebook):
    B, D, H, W = z_e_x.shape
    K = codebook.shape[0]
    HW = H * W
    P = 512
    z = z_e_x.reshape(B, D, HW)
    out = pl.pallas_call(
        _vq_body,
        grid=(B, HW // P),
        in_specs=[
            pl.BlockSpec((1, D, P), lambda b, p: (b, 0, p)),
            pl.BlockSpec((K, D), lambda b, p: (0, 0)),
        ],
        out_specs=pl.BlockSpec((1, 1, P), lambda b, p: (b, 0, p)),
        out_shape=jax.ShapeDtypeStruct((B, 1, HW), jnp.int32),
    )(z, codebook)
    return out.reshape(B, H, W)


# speedup vs baseline: 1.4729x; 1.4729x over previous
import jax
import jax.numpy as jnp
from jax.experimental import pallas as pl
from jax.experimental.pallas import tpu as pltpu


def _vq_body(z_ref, cb_ref, out_ref):
    zb = z_ref[0]
    cb = cb_ref[...]
    K = cb.shape[0]
    cb2 = cb + cb
    mm2 = jax.lax.dot_general(
        cb2, zb, (((1,), (0,)), ((), ())),
        preferred_element_type=jnp.float32)
    x2 = jnp.sum(zb * zb, axis=0, keepdims=True)
    c2 = jnp.sum(cb * cb, axis=1, keepdims=True)
    dist = (x2 - mm2) + c2
    m = jnp.min(dist, axis=0, keepdims=True)
    ks = jax.lax.broadcasted_iota(jnp.int32, (K, 1), 0)
    ksf = ks.astype(jnp.float32)
    idxf = jnp.min(jnp.where(dist == m, ksf, jnp.float32(K)), axis=0)
    out_ref[0, 0, :] = idxf.astype(jnp.int32)


def kernel(z_e_x, codebook):
    B, D, H, W = z_e_x.shape
    K = codebook.shape[0]
    HW = H * W
    P = 1024
    z = z_e_x.reshape(B, D, HW)
    out = pl.pallas_call(
        _vq_body,
        grid=(B, HW // P),
        in_specs=[
            pl.BlockSpec((1, D, P), lambda b, p: (b, 0, p)),
            pl.BlockSpec((K, D), lambda b, p: (0, 0)),
        ],
        out_specs=pl.BlockSpec((1, 1, P), lambda b, p: (b, 0, p)),
        out_shape=jax.ShapeDtypeStruct((B, 1, HW), jnp.int32),
        compiler_params=pltpu.CompilerParams(
            dimension_semantics=("parallel", "parallel")),
    )(z, codebook)
    return out.reshape(B, H, W)
